# Initial kernel scaffold; baseline (speedup 1.0000x reference)
import jax
import jax.numpy as jnp
from jax.experimental import pallas as pl


def kernel(x, gate_W, gate_b, lora_A, lora_B):
    raise NotImplementedError("write your pallas kernel here")


# speedup vs baseline: 6.3055x; 6.3055x over previous
import functools

import jax
import jax.numpy as jnp
from jax.experimental import pallas as pl

_NUM_EXPERTS = 8
_RANK = 16
_TB = 512


def _body(x_ref, gw_ref, gb_ref, a_ref, b_ref, out_ref, probs_ref):
    xb = x_ref[...].astype(jnp.bfloat16)
    logits = jnp.dot(xb, gw_ref[...].astype(jnp.bfloat16),
                     preferred_element_type=jnp.float32) + gb_ref[...]
    mx = jnp.max(logits, axis=-1, keepdims=True)
    ex = jnp.exp(logits - mx)
    probs = ex / jnp.sum(ex, axis=-1, keepdims=True)
    probs_ref[...] = probs

    idx = jax.lax.broadcasted_iota(jnp.int32, probs.shape, 1)
    big = jnp.int32(_NUM_EXPERTS)
    p1 = jnp.max(probs, axis=-1, keepdims=True)
    a1 = jnp.min(jnp.where(probs == p1, idx, big), axis=-1, keepdims=True)
    p_rest = jnp.where(idx == a1, -jnp.inf, probs)
    p2 = jnp.max(p_rest, axis=-1, keepdims=True)
    a2 = jnp.min(jnp.where(p_rest == p2, idx, big), axis=-1, keepdims=True)

    h = jnp.dot(xb, a_ref[...].astype(jnp.bfloat16),
                preferred_element_type=jnp.float32)
    h = 0.5 * h * (1.0 + jax.lax.erf(h * 0.7071067811865476))
    eid = jax.lax.broadcasted_iota(jnp.int32, h.shape, 1) // _RANK
    hm = jnp.where((eid == a1) | (eid == a2), h, 0.0).astype(jnp.bfloat16)
    out_ref[...] = jnp.dot(hm, b_ref[...].astype(jnp.bfloat16),
                           preferred_element_type=jnp.float32)


@functools.partial(jax.jit, static_argnames=())
def kernel(x, gate_W, gate_b, lora_A, lora_B):
    batch, seq, dim = x.shape
    num_experts, rank, _ = lora_A.shape
    n = batch * seq
    hdim = num_experts * rank

    xf = x.reshape(n, dim)
    gw_t = gate_W.T
    gb2 = gate_b.reshape(1, num_experts)
    a_all = lora_A.reshape(hdim, dim).T
    b_all = lora_B.transpose(0, 2, 1).reshape(hdim, dim)

    out_flat, probs_flat = pl.pallas_call(
        _body,
        grid=(n // _TB,),
        in_specs=[
            pl.BlockSpec((_TB, dim), lambda i: (i, 0)),
            pl.BlockSpec((dim, num_experts), lambda i: (0, 0)),
            pl.BlockSpec((1, num_experts), lambda i: (0, 0)),
            pl.BlockSpec((dim, hdim), lambda i: (0, 0)),
            pl.BlockSpec((hdim, dim), lambda i: (0, 0)),
        ],
        out_specs=[
            pl.BlockSpec((_TB, dim), lambda i: (i, 0)),
            pl.BlockSpec((_TB, num_experts), lambda i: (i, 0)),
        ],
        out_shape=[
            jax.ShapeDtypeStruct((n, dim), jnp.float32),
            jax.ShapeDtypeStruct((n, num_experts), jnp.float32),
        ],
    )(xf, gw_t, gb2, a_all, b_all)
    return out_flat.reshape(batch, seq, dim), probs_flat.reshape(batch, seq, num_experts)
